# baseline (speedup 1.0000x reference)
import functools

import jax
import jax.numpy as jnp
from jax import lax
from jax.experimental import pallas as pl
from jax.experimental.pallas import tpu as pltpu
from jax.experimental.pallas import tpu_sc as plsc

N_NODES = 10000
N_EDGES = 320000
D = 128
N_BASIS = 16
R_MAX = 5.0

NC = 2
NS = 16
NW = NC * NS
EDGES_PER_TILE = N_EDGES // NW
CHUNK = 40
N_CHUNKS = EDGES_PER_TILE // CHUNK
DUMP_ROWS = 624
TAIL_ROWS = N_NODES - NS * DUMP_ROWS
ZROWS = 52

NODE_BLK = 1000
EDGE_BLK = 3200


def _silu(v):
    return v * (1.0 / (1.0 + jnp.exp(-v)))


def _pack_bf16_pairs(a, b):
    ua = lax.bitcast_convert_type(
        a.astype(jnp.bfloat16).astype(jnp.float32), jnp.int32)
    ub = lax.bitcast_convert_type(
        b.astype(jnp.bfloat16).astype(jnp.float32), jnp.int32)
    return (ub & (-65536)) | lax.shift_right_logical(ua, 16)


def _node_embed_body(x_ref, w1_ref, b1_ref, w2_ref, b2_ref, h_ref, hp_ref):
    t = jnp.dot(x_ref[...], w1_ref[...], preferred_element_type=jnp.float32)
    t = _silu(t + b1_ref[...])
    h = jnp.dot(t, w2_ref[...], preferred_element_type=jnp.float32) + b2_ref[...]
    h_ref[...] = h
    hp = _pack_bf16_pairs(h[:, : D // 2], h[:, D // 2 :])
    hp_ref[...] = jnp.concatenate([hp, jnp.zeros_like(hp)], axis=1)


def _node_embed(x, w1, b1, w2, b2):
    grid = (N_NODES // NODE_BLK,)
    return pl.pallas_call(
        _node_embed_body,
        grid=grid,
        in_specs=[
            pl.BlockSpec((NODE_BLK, D), lambda i: (i, 0)),
            pl.BlockSpec((D, D), lambda i: (0, 0)),
            pl.BlockSpec((1, D), lambda i: (0, 0)),
            pl.BlockSpec((D, D), lambda i: (0, 0)),
            pl.BlockSpec((1, D), lambda i: (0, 0)),
        ],
        out_specs=[
            pl.BlockSpec((NODE_BLK, D), lambda i: (i, 0)),
            pl.BlockSpec((NODE_BLK, D), lambda i: (i, 0)),
        ],
        out_shape=[
            jax.ShapeDtypeStruct((N_NODES, D), jnp.float32),
            jax.ShapeDtypeStruct((N_NODES, D), jnp.int32),
        ],
    )(x, w1, b1, w2, b2)


def _radial_body(evt_ref, wr1_ref, br1_ref, wr2_ref, br2_ref, wr3_ref, w_ref):
    evt = evt_ref[...]
    r2 = jnp.sum(evt * evt, axis=0, keepdims=True) + 1e-12
    r = jnp.sqrt(r2)
    ru = jnp.clip(r * (1.0 / R_MAX), 0.0, 1.0)
    cutoff = 0.5 * (jnp.cos(jnp.pi * ru) + 1.0)
    step = R_MAX / (N_BASIS - 1)
    centers = lax.broadcasted_iota(jnp.int32, (N_BASIS, 1), 0).astype(
        jnp.float32) * step
    width = R_MAX / N_BASIS
    diff = jnp.broadcast_to(r, (N_BASIS, r.shape[1])) - centers
    basis_t = jnp.exp(-(diff * diff) * (1.0 / (width * width)))
    basis_t = basis_t * cutoff
    t = _silu(
        lax.dot_general(
            basis_t, wr1_ref[...],
            dimension_numbers=(((0,), (0,)), ((), ())),
            preferred_element_type=jnp.float32,
        )
        + br1_ref[...]
    )
    t = _silu(
        jnp.dot(t, wr2_ref[...], preferred_element_type=jnp.float32) + br2_ref[...]
    )
    wf = jnp.dot(t, wr3_ref[...], preferred_element_type=jnp.float32)
    w_ref[...] = _pack_bf16_pairs(wf[:, : D // 2], wf[:, D // 2 :])


def _radial(edge_vec_t, n_edges, wr1, br1, wr2, br2, wr3):
    grid = (n_edges // EDGE_BLK,)
    return pl.pallas_call(
        _radial_body,
        grid=grid,
        in_specs=[
            pl.BlockSpec((3, EDGE_BLK), lambda i: (0, i)),
            pl.BlockSpec((N_BASIS, 64), lambda i: (0, 0)),
            pl.BlockSpec((1, 64), lambda i: (0, 0)),
            pl.BlockSpec((64, 64), lambda i: (0, 0)),
            pl.BlockSpec((1, 64), lambda i: (0, 0)),
            pl.BlockSpec((64, D), lambda i: (0, 0)),
        ],
        out_specs=pl.BlockSpec((EDGE_BLK, D // 2), lambda i: (i, 0)),
        out_shape=jax.ShapeDtypeStruct((n_edges, D // 2), jnp.int32),
    )(edge_vec_t, wr1, br1, wr2, br2, wr3)


_SC_MESH = plsc.VectorSubcoreMesh(
    core_axis_name="c", subcore_axis_name="s", num_cores=NC, num_subcores=NS
)


NIDX = 5
NROW = 2
GROUP = 10


def _edge_aggregate_body(edges_per_tile, n_chunks,
                         h_hbm, w_hbm, src_hbm, dst_hbm, out_hbm, *scr):
    srcb = scr[0:NIDX]
    dstb = scr[NIDX:2 * NIDX]
    rows = scr[2 * NIDX:2 * NIDX + NROW]
    wc = scr[2 * NIDX + NROW:2 * NIDX + 2 * NROW]
    msg = scr[2 * NIDX + 2 * NROW:2 * NIDX + 3 * NROW]
    z_v = scr[2 * NIDX + 3 * NROW]
    agg_sh = scr[2 * NIDX + 3 * NROW + 1]
    sems = scr[2 * NIDX + 3 * NROW + 2:]
    sem_si = sems[0:NIDX]
    sem_di = sems[NIDX:2 * NIDX]
    sem_g = sems[2 * NIDX:2 * NIDX + NROW]
    sem_w = sems[2 * NIDX + NROW:2 * NIDX + 2 * NROW]
    sem_s = sems[2 * NIDX + 2 * NROW:2 * NIDX + 3 * NROW]

    c = lax.axis_index("c")
    s = lax.axis_index("s")
    wid = c * NS + s
    ebase = wid * edges_per_tile

    def start_src(chunk_i, b):
        pltpu.async_copy(
            src_hbm.at[pl.ds(ebase + chunk_i * CHUNK, CHUNK)], srcb[b],
            sem_si[b])

    def wait_src(b):
        pltpu.make_async_copy(
            src_hbm.at[pl.ds(ebase, CHUNK)], srcb[b], sem_si[b]).wait()

    def start_dst(chunk_i, b):
        pltpu.async_copy(
            dst_hbm.at[pl.ds(ebase + chunk_i * CHUNK, CHUNK)], dstb[b],
            sem_di[b])

    def wait_dst(b):
        pltpu.make_async_copy(
            dst_hbm.at[pl.ds(ebase, CHUNK)], dstb[b], sem_di[b]).wait()

    def start_gather(bi, br):
        pltpu.async_copy(h_hbm.at[srcb[bi]], rows[br], sem_g[br])

    def wait_gather(bi, br):
        pltpu.make_async_copy(h_hbm.at[srcb[bi]], rows[br], sem_g[br]).wait()

    def start_w(chunk_i, b):
        pltpu.async_copy(
            w_hbm.at[pl.ds(ebase + chunk_i * CHUNK, CHUNK)], wc[b], sem_w[b])

    def wait_w(b):
        pltpu.make_async_copy(
            w_hbm.at[pl.ds(ebase, CHUNK)], wc[b], sem_w[b]).wait()

    def start_scatter(bi, bm):
        pltpu.async_copy(msg[bm], agg_sh.at[dstb[bi]], sem_s[bm], add=True)

    def wait_scatter(bi, bm):
        pltpu.make_async_copy(msg[bm], agg_sh.at[dstb[bi]], sem_s[bm]).wait()

    def multiply(b):
        hi_mask = jnp.full((16,), -65536, jnp.int32)
        lo_shift = jnp.full((16,), 65536, jnp.int32)

        def _mrow(r, cc):
            for j in range(D // 32):
                hu = rows[b][r, pl.ds(j * 16, 16)]
                wu = wc[b][r, pl.ds(j * 16, 16)]
                ha = lax.bitcast_convert_type(hu * lo_shift, jnp.float32)
                wa = lax.bitcast_convert_type(wu * lo_shift, jnp.float32)
                hb = lax.bitcast_convert_type(hu & hi_mask, jnp.float32)
                wb = lax.bitcast_convert_type(wu & hi_mask, jnp.float32)
                msg[b][r, pl.ds(j * 16, 16)] = ha * wa
                msg[b][r, pl.ds(D // 2 + j * 16, 16)] = hb * wb
            return cc

        lax.fori_loop(0, CHUNK, _mrow, 0)

    def _zrow(i, carry):
        for j in range(D // 16):
            z_v[i, pl.ds(j * 16, 16)] = jnp.zeros((16,), jnp.float32)
        return carry

    lax.fori_loop(0, ZROWS, _zrow, 0)
    row0 = s * DUMP_ROWS
    for j in range(DUMP_ROWS // ZROWS):
        pltpu.sync_copy(z_v, agg_sh.at[pl.ds(row0 + j * ZROWS, ZROWS)])

    @pl.when(s == NS - 1)
    def _zero_tail():
        pltpu.sync_copy(
            z_v.at[pl.ds(0, TAIL_ROWS)],
            agg_sh.at[pl.ds(NS * DUMP_ROWS, TAIL_ROWS)],
        )

    plsc.subcore_barrier()

    def step_a(j, g=None):
        bi, br = j % NIDX, j % NROW
        wait_gather(bi, br)
        wait_w(br)
        if g is None or j >= NROW:
            wait_scatter((j - NROW) % NIDX, br)
        else:
            @pl.when(g > 0)
            def _ws():
                wait_scatter((j - NROW) % NIDX, br)
        wait_dst(bi)
        multiply(br)
        start_scatter(bi, br)

    def step_b(j, i2):
        bi2, br = (j + 2) % NIDX, j % NROW
        wait_src(bi2)
        start_dst(i2, bi2)
        start_gather(bi2, br)
        start_w(i2, br)

    for k in range(NIDX):
        start_src(k, k)
    for k in range(NROW):
        wait_src(k)
        start_dst(k, k)
        start_gather(k, k)
        start_w(k, k)

    if n_chunks % GROUP == 0:
        n_main, ep = n_chunks // GROUP - 1, GROUP
    else:
        n_main, ep = n_chunks // GROUP, n_chunks % GROUP
    assert ep >= NIDX

    def _group(g, carry):
        for j in range(GROUP):
            i = g * GROUP + j
            step_a(j, g)
            step_b(j, i + 2)
            start_src(i + NIDX, j % NIDX)
        return carry

    lax.fori_loop(0, n_main, _group, 0)

    i0 = n_main * GROUP
    for j in range(ep):
        step_a(j)
        if i0 + j + 2 < n_chunks:
            step_b(j, i0 + j + 2)
        if i0 + j + NIDX < n_chunks:
            start_src(i0 + j + NIDX, j % NIDX)
    for bm in range(NROW):
        wait_scatter((ep - NROW + bm) % NIDX, bm)

    plsc.subcore_barrier()

    pltpu.sync_copy(
        agg_sh.at[pl.ds(row0, DUMP_ROWS)],
        out_hbm.at[c, pl.ds(row0, DUMP_ROWS)],
    )

    @pl.when(s == NS - 1)
    def _dump_tail():
        pltpu.sync_copy(
            agg_sh.at[pl.ds(NS * DUMP_ROWS, TAIL_ROWS)],
            out_hbm.at[c, pl.ds(NS * DUMP_ROWS, TAIL_ROWS)],
        )


@functools.lru_cache(maxsize=4)
def _edge_aggregate_kernel(n_edges):
    edges_per_tile = n_edges // NW
    n_chunks = edges_per_tile // CHUNK
    scratch = (
        [pltpu.VMEM((CHUNK,), jnp.int32) for _ in range(NIDX)]
        + [pltpu.VMEM((CHUNK,), jnp.int32) for _ in range(NIDX)]
        + [pltpu.VMEM((CHUNK, D), jnp.int32) for _ in range(NROW)]
        + [pltpu.VMEM((CHUNK, D // 2), jnp.int32) for _ in range(NROW)]
        + [pltpu.VMEM((CHUNK, D), jnp.float32) for _ in range(NROW)]
        + [pltpu.VMEM((ZROWS, D), jnp.float32)]
        + [pltpu.VMEM_SHARED((N_NODES, D), jnp.float32)]
        + [pltpu.SemaphoreType.DMA for _ in range(2 * NIDX + 3 * NROW)]
    )
    return pl.kernel(
        functools.partial(_edge_aggregate_body, edges_per_tile, n_chunks),
        out_type=jax.ShapeDtypeStruct((NC, N_NODES, D), jnp.float32),
        mesh=_SC_MESH,
        scratch_types=scratch,
    )


def _combine_body(a0_ref, a1_ref, a2_ref, a3_ref, h_ref, ws_ref, wa_ref,
                  wo_ref, bo_ref, o_ref):
    agg = (a0_ref[...] + a1_ref[...]) + (a2_ref[...] + a3_ref[...])
    t = jnp.dot(agg, ws_ref[...], preferred_element_type=jnp.float32) + jnp.dot(
        h_ref[...], wa_ref[...], preferred_element_type=jnp.float32
    )
    t = _silu(t)
    o_ref[...] = _silu(
        jnp.dot(t, wo_ref[...], preferred_element_type=jnp.float32) + bo_ref[...]
    )


def _combine(a0, a1, a2, a3, h, w_self, w_attr, w_out, b_out):
    grid = (N_NODES // NODE_BLK,)
    blk = lambda i: (i, 0)
    full = lambda i: (0, 0)
    return pl.pallas_call(
        _combine_body,
        grid=grid,
        in_specs=[
            pl.BlockSpec((NODE_BLK, D), blk),
            pl.BlockSpec((NODE_BLK, D), blk),
            pl.BlockSpec((NODE_BLK, D), blk),
            pl.BlockSpec((NODE_BLK, D), blk),
            pl.BlockSpec((NODE_BLK, D), blk),
            pl.BlockSpec((D, D), full),
            pl.BlockSpec((D, D), full),
            pl.BlockSpec((D, D), full),
            pl.BlockSpec((1, D), full),
        ],
        out_specs=pl.BlockSpec((NODE_BLK, D), blk),
        out_shape=jax.ShapeDtypeStruct((N_NODES, D), jnp.float32),
    )(a0, a1, a2, a3, h, w_self, w_attr, w_out, b_out)


def kernel(x, edge_index, edge_vec, W_emb1, b_emb1, W_emb2, b_emb2,
           Wr1, br1, Wr2, br2, Wr3, W_self, W_attr, W_out, b_out):
    h, hp = _node_embed(
        x, W_emb1, b_emb1.reshape(1, D), W_emb2, b_emb2.reshape(1, D))
    ei = edge_index.astype(jnp.int32)
    evt = edge_vec.T
    e2 = N_EDGES // 2
    br1r = br1.reshape(1, 64)
    br2r = br2.reshape(1, 64)
    w_a = _radial(evt[:, :e2], e2, Wr1, br1r, Wr2, br2r, Wr3)
    parts_a = _edge_aggregate_kernel(e2)(hp, w_a, ei[0, :e2], ei[1, :e2])
    w_b = _radial(evt[:, e2:], e2, Wr1, br1r, Wr2, br2r, Wr3)
    parts_b = _edge_aggregate_kernel(e2)(hp, w_b, ei[0, e2:], ei[1, e2:])
    return _combine(parts_a[0], parts_a[1], parts_b[0], parts_b[1], h,
                    W_self, W_attr, W_out, b_out.reshape(1, D))


# speedup vs baseline: 1.0401x; 1.0401x over previous
import functools

import jax
import jax.numpy as jnp
from jax import lax
from jax.experimental import pallas as pl
from jax.experimental.pallas import tpu as pltpu
from jax.experimental.pallas import tpu_sc as plsc

N_NODES = 10000
N_EDGES = 320000
D = 128
N_BASIS = 16
R_MAX = 5.0

NC = 2
NS = 16
NW = NC * NS
EDGES_PER_TILE = N_EDGES // NW
CHUNK = 40
N_CHUNKS = EDGES_PER_TILE // CHUNK
DUMP_ROWS = 624
TAIL_ROWS = N_NODES - NS * DUMP_ROWS
ZROWS = 52

NODE_BLK = 1000
EDGE_BLK = 3200


def _silu(v):
    return v * (1.0 / (1.0 + jnp.exp(-v)))


def _pack_bf16_pairs(a, b):
    ua = lax.bitcast_convert_type(
        a.astype(jnp.bfloat16).astype(jnp.float32), jnp.int32)
    ub = lax.bitcast_convert_type(
        b.astype(jnp.bfloat16).astype(jnp.float32), jnp.int32)
    return (ub & (-65536)) | lax.shift_right_logical(ua, 16)


def _node_embed_body(x_ref, w1_ref, b1_ref, w2_ref, b2_ref, h_ref):
    t = jnp.dot(x_ref[...], w1_ref[...], preferred_element_type=jnp.float32)
    t = _silu(t + b1_ref[...])
    h_ref[...] = (
        jnp.dot(t, w2_ref[...], preferred_element_type=jnp.float32) + b2_ref[...]
    )


def _node_embed(x, w1, b1, w2, b2):
    grid = (N_NODES // NODE_BLK,)
    return pl.pallas_call(
        _node_embed_body,
        grid=grid,
        in_specs=[
            pl.BlockSpec((NODE_BLK, D), lambda i: (i, 0)),
            pl.BlockSpec((D, D), lambda i: (0, 0)),
            pl.BlockSpec((1, D), lambda i: (0, 0)),
            pl.BlockSpec((D, D), lambda i: (0, 0)),
            pl.BlockSpec((1, D), lambda i: (0, 0)),
        ],
        out_specs=pl.BlockSpec((NODE_BLK, D), lambda i: (i, 0)),
        out_shape=jax.ShapeDtypeStruct((N_NODES, D), jnp.float32),
    )(x, w1, b1, w2, b2)


def _radial_body(evt_ref, wr1_ref, br1_ref, wr2_ref, br2_ref, wr3_ref, w_ref):
    evt = evt_ref[...]
    r2 = jnp.sum(evt * evt, axis=0, keepdims=True) + 1e-12
    r = jnp.sqrt(r2)
    ru = jnp.clip(r * (1.0 / R_MAX), 0.0, 1.0)
    cutoff = 0.5 * (jnp.cos(jnp.pi * ru) + 1.0)
    step = R_MAX / (N_BASIS - 1)
    centers = lax.broadcasted_iota(jnp.int32, (N_BASIS, 1), 0).astype(
        jnp.float32) * step
    width = R_MAX / N_BASIS
    diff = jnp.broadcast_to(r, (N_BASIS, r.shape[1])) - centers
    basis_t = jnp.exp(-(diff * diff) * (1.0 / (width * width)))
    basis_t = basis_t * cutoff
    t = _silu(
        lax.dot_general(
            basis_t, wr1_ref[...],
            dimension_numbers=(((0,), (0,)), ((), ())),
            preferred_element_type=jnp.float32,
        )
        + br1_ref[...]
    )
    t = _silu(
        jnp.dot(t, wr2_ref[...], preferred_element_type=jnp.float32) + br2_ref[...]
    )
    w_ref[...] = jnp.dot(t, wr3_ref[...], preferred_element_type=jnp.float32)


def _radial(edge_vec_t, n_edges, wr1, br1, wr2, br2, wr3):
    grid = (n_edges // EDGE_BLK,)
    return pl.pallas_call(
        _radial_body,
        grid=grid,
        in_specs=[
            pl.BlockSpec((3, EDGE_BLK), lambda i: (0, i)),
            pl.BlockSpec((N_BASIS, 64), lambda i: (0, 0)),
            pl.BlockSpec((1, 64), lambda i: (0, 0)),
            pl.BlockSpec((64, 64), lambda i: (0, 0)),
            pl.BlockSpec((1, 64), lambda i: (0, 0)),
            pl.BlockSpec((64, D), lambda i: (0, 0)),
        ],
        out_specs=pl.BlockSpec((EDGE_BLK, D), lambda i: (i, 0)),
        out_shape=jax.ShapeDtypeStruct((n_edges, D), jnp.float32),
    )(edge_vec_t, wr1, br1, wr2, br2, wr3)


_SC_MESH = plsc.VectorSubcoreMesh(
    core_axis_name="c", subcore_axis_name="s", num_cores=NC, num_subcores=NS
)


NIDX = 5
NROW = 2
GROUP = 10


def _edge_aggregate_body(edges_per_tile, n_chunks,
                         h_hbm, w_hbm, src_hbm, dst_hbm, out_hbm, *scr):
    srcb = scr[0:NIDX]
    dstb = scr[NIDX:2 * NIDX]
    rows = scr[2 * NIDX:2 * NIDX + NROW]
    wc = scr[2 * NIDX + NROW:2 * NIDX + 2 * NROW]
    msg = scr[2 * NIDX + 2 * NROW:2 * NIDX + 3 * NROW]
    z_v = scr[2 * NIDX + 3 * NROW]
    agg_sh = scr[2 * NIDX + 3 * NROW + 1]
    sems = scr[2 * NIDX + 3 * NROW + 2:]
    sem_si = sems[0:NIDX]
    sem_di = sems[NIDX:2 * NIDX]
    sem_g = sems[2 * NIDX:2 * NIDX + NROW]
    sem_w = sems[2 * NIDX + NROW:2 * NIDX + 2 * NROW]
    sem_s = sems[2 * NIDX + 2 * NROW:2 * NIDX + 3 * NROW]

    c = lax.axis_index("c")
    s = lax.axis_index("s")
    wid = c * NS + s
    ebase = wid * edges_per_tile

    def start_src(chunk_i, b):
        pltpu.async_copy(
            src_hbm.at[pl.ds(ebase + chunk_i * CHUNK, CHUNK)], srcb[b],
            sem_si[b])

    def wait_src(b):
        pltpu.make_async_copy(
            src_hbm.at[pl.ds(ebase, CHUNK)], srcb[b], sem_si[b]).wait()

    def start_dst(chunk_i, b):
        pltpu.async_copy(
            dst_hbm.at[pl.ds(ebase + chunk_i * CHUNK, CHUNK)], dstb[b],
            sem_di[b])

    def wait_dst(b):
        pltpu.make_async_copy(
            dst_hbm.at[pl.ds(ebase, CHUNK)], dstb[b], sem_di[b]).wait()

    def start_gather(bi, br):
        pltpu.async_copy(h_hbm.at[srcb[bi]], rows[br], sem_g[br])

    def wait_gather(bi, br):
        pltpu.make_async_copy(h_hbm.at[srcb[bi]], rows[br], sem_g[br]).wait()

    def start_w(chunk_i, b):
        pltpu.async_copy(
            w_hbm.at[pl.ds(ebase + chunk_i * CHUNK, CHUNK)], wc[b], sem_w[b])

    def wait_w(b):
        pltpu.make_async_copy(
            w_hbm.at[pl.ds(ebase, CHUNK)], wc[b], sem_w[b]).wait()

    def start_scatter(bi, bm):
        pltpu.async_copy(msg[bm], agg_sh.at[dstb[bi]], sem_s[bm], add=True)

    def wait_scatter(bi, bm):
        pltpu.make_async_copy(msg[bm], agg_sh.at[dstb[bi]], sem_s[bm]).wait()

    def multiply(b):
        def _mrow(r):
            for j in range(D // 16):
                sl = pl.ds(j * 16, 16)
                msg[b][r, sl] = rows[b][r, sl] * wc[b][r, sl]

        plsc.parallel_loop(0, CHUNK, 1, unroll=2)(_mrow)

    def _zrow(i, carry):
        for j in range(D // 16):
            z_v[i, pl.ds(j * 16, 16)] = jnp.zeros((16,), jnp.float32)
        return carry

    lax.fori_loop(0, ZROWS, _zrow, 0)
    row0 = s * DUMP_ROWS
    for j in range(DUMP_ROWS // ZROWS):
        pltpu.sync_copy(z_v, agg_sh.at[pl.ds(row0 + j * ZROWS, ZROWS)])

    @pl.when(s == NS - 1)
    def _zero_tail():
        pltpu.sync_copy(
            z_v.at[pl.ds(0, TAIL_ROWS)],
            agg_sh.at[pl.ds(NS * DUMP_ROWS, TAIL_ROWS)],
        )

    plsc.subcore_barrier()

    def step_a(j, g=None):
        bi, br = j % NIDX, j % NROW
        wait_gather(bi, br)
        wait_w(br)
        if g is None or j >= NROW:
            wait_scatter((j - NROW) % NIDX, br)
        else:
            @pl.when(g > 0)
            def _ws():
                wait_scatter((j - NROW) % NIDX, br)
        wait_dst(bi)
        multiply(br)
        start_scatter(bi, br)

    def step_b(j, i2):
        bi2, br = (j + 2) % NIDX, j % NROW
        wait_src(bi2)
        start_dst(i2, bi2)
        start_gather(bi2, br)
        start_w(i2, br)

    for k in range(NIDX):
        start_src(k, k)
    for k in range(NROW):
        wait_src(k)
        start_dst(k, k)
        start_gather(k, k)
        start_w(k, k)

    if n_chunks % GROUP == 0:
        n_main, ep = n_chunks // GROUP - 1, GROUP
    else:
        n_main, ep = n_chunks // GROUP, n_chunks % GROUP
    assert ep >= NIDX

    def _group(g, carry):
        for j in range(GROUP):
            i = g * GROUP + j
            step_a(j, g)
            step_b(j, i + 2)
            start_src(i + NIDX, j % NIDX)
        return carry

    lax.fori_loop(0, n_main, _group, 0)

    i0 = n_main * GROUP
    for j in range(ep):
        step_a(j)
        if i0 + j + 2 < n_chunks:
            step_b(j, i0 + j + 2)
        if i0 + j + NIDX < n_chunks:
            start_src(i0 + j + NIDX, j % NIDX)
    for bm in range(NROW):
        wait_scatter((ep - NROW + bm) % NIDX, bm)

    plsc.subcore_barrier()

    pltpu.sync_copy(
        agg_sh.at[pl.ds(row0, DUMP_ROWS)],
        out_hbm.at[c, pl.ds(row0, DUMP_ROWS)],
    )

    @pl.when(s == NS - 1)
    def _dump_tail():
        pltpu.sync_copy(
            agg_sh.at[pl.ds(NS * DUMP_ROWS, TAIL_ROWS)],
            out_hbm.at[c, pl.ds(NS * DUMP_ROWS, TAIL_ROWS)],
        )


@functools.lru_cache(maxsize=4)
def _edge_aggregate_kernel(n_edges):
    edges_per_tile = n_edges // NW
    n_chunks = edges_per_tile // CHUNK
    scratch = (
        [pltpu.VMEM((CHUNK,), jnp.int32) for _ in range(NIDX)]
        + [pltpu.VMEM((CHUNK,), jnp.int32) for _ in range(NIDX)]
        + [pltpu.VMEM((CHUNK, D), jnp.float32) for _ in range(NROW)]
        + [pltpu.VMEM((CHUNK, D), jnp.float32) for _ in range(NROW)]
        + [pltpu.VMEM((CHUNK, D), jnp.float32) for _ in range(NROW)]
        + [pltpu.VMEM((ZROWS, D), jnp.float32)]
        + [pltpu.VMEM_SHARED((N_NODES, D), jnp.float32)]
        + [pltpu.SemaphoreType.DMA for _ in range(2 * NIDX + 3 * NROW)]
    )
    return pl.kernel(
        functools.partial(_edge_aggregate_body, edges_per_tile, n_chunks),
        out_type=jax.ShapeDtypeStruct((NC, N_NODES, D), jnp.float32),
        mesh=_SC_MESH,
        scratch_types=scratch,
    )


def _combine_body(a0_ref, a1_ref, a2_ref, a3_ref, h_ref, ws_ref, wa_ref,
                  wo_ref, bo_ref, o_ref):
    agg = (a0_ref[...] + a1_ref[...]) + (a2_ref[...] + a3_ref[...])
    t = jnp.dot(agg, ws_ref[...], preferred_element_type=jnp.float32) + jnp.dot(
        h_ref[...], wa_ref[...], preferred_element_type=jnp.float32
    )
    t = _silu(t)
    o_ref[...] = _silu(
        jnp.dot(t, wo_ref[...], preferred_element_type=jnp.float32) + bo_ref[...]
    )


def _combine(a0, a1, a2, a3, h, w_self, w_attr, w_out, b_out):
    grid = (N_NODES // NODE_BLK,)
    blk = lambda i: (i, 0)
    full = lambda i: (0, 0)
    return pl.pallas_call(
        _combine_body,
        grid=grid,
        in_specs=[
            pl.BlockSpec((NODE_BLK, D), blk),
            pl.BlockSpec((NODE_BLK, D), blk),
            pl.BlockSpec((NODE_BLK, D), blk),
            pl.BlockSpec((NODE_BLK, D), blk),
            pl.BlockSpec((NODE_BLK, D), blk),
            pl.BlockSpec((D, D), full),
            pl.BlockSpec((D, D), full),
            pl.BlockSpec((D, D), full),
            pl.BlockSpec((1, D), full),
        ],
        out_specs=pl.BlockSpec((NODE_BLK, D), blk),
        out_shape=jax.ShapeDtypeStruct((N_NODES, D), jnp.float32),
    )(a0, a1, a2, a3, h, w_self, w_attr, w_out, b_out)


def kernel(x, edge_index, edge_vec, W_emb1, b_emb1, W_emb2, b_emb2,
           Wr1, br1, Wr2, br2, Wr3, W_self, W_attr, W_out, b_out):
    h = _node_embed(x, W_emb1, b_emb1.reshape(1, D), W_emb2, b_emb2.reshape(1, D))
    ei = edge_index.astype(jnp.int32)
    evt = edge_vec.T
    e2 = N_EDGES // 2
    br1r = br1.reshape(1, 64)
    br2r = br2.reshape(1, 64)
    w_a = _radial(evt[:, :e2], e2, Wr1, br1r, Wr2, br2r, Wr3)
    parts_a = _edge_aggregate_kernel(e2)(h, w_a, ei[0, :e2], ei[1, :e2])
    w_b = _radial(evt[:, e2:], e2, Wr1, br1r, Wr2, br2r, Wr3)
    parts_b = _edge_aggregate_kernel(e2)(h, w_b, ei[0, e2:], ei[1, e2:])
    return _combine(parts_a[0], parts_a[1], parts_b[0], parts_b[1], h,
                    W_self, W_attr, W_out, b_out.reshape(1, D))


# speedup vs baseline: 1.0749x; 1.0334x over previous
import functools

import jax
import jax.numpy as jnp
from jax import lax
from jax.experimental import pallas as pl
from jax.experimental.pallas import tpu as pltpu
from jax.experimental.pallas import tpu_sc as plsc

N_NODES = 10000
N_EDGES = 320000
D = 128
N_BASIS = 16
R_MAX = 5.0

NC = 2
NS = 16
NW = NC * NS
EDGES_PER_TILE = N_EDGES // NW
CHUNK = 40
N_CHUNKS = EDGES_PER_TILE // CHUNK
DUMP_ROWS = 624
TAIL_ROWS = N_NODES - NS * DUMP_ROWS
ZROWS = 52

NODE_BLK = 1000
EDGE_BLK = 3200


def _silu(v):
    return v * (1.0 / (1.0 + jnp.exp(-v)))


def _pack_bf16_pairs(a, b):
    ua = lax.bitcast_convert_type(
        a.astype(jnp.bfloat16).astype(jnp.float32), jnp.int32)
    ub = lax.bitcast_convert_type(
        b.astype(jnp.bfloat16).astype(jnp.float32), jnp.int32)
    return (ub & (-65536)) | lax.shift_right_logical(ua, 16)


def _node_embed_body(x_ref, w1_ref, b1_ref, w2_ref, b2_ref, h_ref):
    t = jnp.dot(x_ref[...], w1_ref[...], preferred_element_type=jnp.float32)
    t = _silu(t + b1_ref[...])
    h_ref[...] = (
        jnp.dot(t, w2_ref[...], preferred_element_type=jnp.float32) + b2_ref[...]
    )


def _node_embed(x, w1, b1, w2, b2):
    grid = (N_NODES // NODE_BLK,)
    return pl.pallas_call(
        _node_embed_body,
        grid=grid,
        in_specs=[
            pl.BlockSpec((NODE_BLK, D), lambda i: (i, 0)),
            pl.BlockSpec((D, D), lambda i: (0, 0)),
            pl.BlockSpec((1, D), lambda i: (0, 0)),
            pl.BlockSpec((D, D), lambda i: (0, 0)),
            pl.BlockSpec((1, D), lambda i: (0, 0)),
        ],
        out_specs=pl.BlockSpec((NODE_BLK, D), lambda i: (i, 0)),
        out_shape=jax.ShapeDtypeStruct((N_NODES, D), jnp.float32),
    )(x, w1, b1, w2, b2)


def _radial_body(evt_ref, wr1_ref, br1_ref, wr2_ref, br2_ref, wr3_ref, w_ref):
    evt = evt_ref[...]
    r2 = jnp.sum(evt * evt, axis=0, keepdims=True) + 1e-12
    r = jnp.sqrt(r2)
    ru = jnp.clip(r * (1.0 / R_MAX), 0.0, 1.0)
    cutoff = 0.5 * (jnp.cos(jnp.pi * ru) + 1.0)
    step = R_MAX / (N_BASIS - 1)
    centers = lax.broadcasted_iota(jnp.int32, (N_BASIS, 1), 0).astype(
        jnp.float32) * step
    width = R_MAX / N_BASIS
    diff = jnp.broadcast_to(r, (N_BASIS, r.shape[1])) - centers
    basis_t = jnp.exp(-(diff * diff) * (1.0 / (width * width)))
    basis_t = basis_t * cutoff
    t = _silu(
        lax.dot_general(
            basis_t, wr1_ref[...],
            dimension_numbers=(((0,), (0,)), ((), ())),
            preferred_element_type=jnp.float32,
        )
        + br1_ref[...]
    )
    t = _silu(
        jnp.dot(t, wr2_ref[...], preferred_element_type=jnp.float32) + br2_ref[...]
    )
    w_ref[...] = jnp.dot(t, wr3_ref[...], preferred_element_type=jnp.float32)


def _radial(edge_vec_t, n_edges, wr1, br1, wr2, br2, wr3):
    grid = (n_edges // EDGE_BLK,)
    return pl.pallas_call(
        _radial_body,
        grid=grid,
        in_specs=[
            pl.BlockSpec((3, EDGE_BLK), lambda i: (0, i)),
            pl.BlockSpec((N_BASIS, 64), lambda i: (0, 0)),
            pl.BlockSpec((1, 64), lambda i: (0, 0)),
            pl.BlockSpec((64, 64), lambda i: (0, 0)),
            pl.BlockSpec((1, 64), lambda i: (0, 0)),
            pl.BlockSpec((64, D), lambda i: (0, 0)),
        ],
        out_specs=pl.BlockSpec((EDGE_BLK, D), lambda i: (i, 0)),
        out_shape=jax.ShapeDtypeStruct((n_edges, D), jnp.float32),
    )(edge_vec_t, wr1, br1, wr2, br2, wr3)


_SC_MESH = plsc.VectorSubcoreMesh(
    core_axis_name="c", subcore_axis_name="s", num_cores=NC, num_subcores=NS
)


NIDX = 5
NROW = 2
GROUP = 10


def _edge_aggregate_body(edges_per_tile, n_chunks,
                         h_hbm, w_hbm, src_hbm, dst_hbm, out_hbm, *scr):
    srcb = scr[0:NIDX]
    dstb = scr[NIDX:2 * NIDX]
    rows = scr[2 * NIDX:2 * NIDX + NROW]
    wc = scr[2 * NIDX + NROW:2 * NIDX + 2 * NROW]
    msg = scr[2 * NIDX + 2 * NROW:2 * NIDX + 3 * NROW]
    z_v = scr[2 * NIDX + 3 * NROW]
    agg_sh = scr[2 * NIDX + 3 * NROW + 1]
    sems = scr[2 * NIDX + 3 * NROW + 2:]
    sem_si = sems[0:NIDX]
    sem_di = sems[NIDX:2 * NIDX]
    sem_g = sems[2 * NIDX:2 * NIDX + NROW]
    sem_w = sems[2 * NIDX + NROW:2 * NIDX + 2 * NROW]
    sem_s = sems[2 * NIDX + 2 * NROW:2 * NIDX + 3 * NROW]

    c = lax.axis_index("c")
    s = lax.axis_index("s")
    wid = c * NS + s
    ebase = wid * edges_per_tile

    def start_src(chunk_i, b):
        pltpu.async_copy(
            src_hbm.at[pl.ds(ebase + chunk_i * CHUNK, CHUNK)], srcb[b],
            sem_si[b])

    def wait_src(b):
        pltpu.make_async_copy(
            src_hbm.at[pl.ds(ebase, CHUNK)], srcb[b], sem_si[b]).wait()

    def start_dst(chunk_i, b):
        pltpu.async_copy(
            dst_hbm.at[pl.ds(ebase + chunk_i * CHUNK, CHUNK)], dstb[b],
            sem_di[b])

    def wait_dst(b):
        pltpu.make_async_copy(
            dst_hbm.at[pl.ds(ebase, CHUNK)], dstb[b], sem_di[b]).wait()

    def start_gather(bi, br):
        pltpu.async_copy(h_hbm.at[srcb[bi]], rows[br], sem_g[br])

    def wait_gather(bi, br):
        pltpu.make_async_copy(h_hbm.at[srcb[bi]], rows[br], sem_g[br]).wait()

    def start_w(chunk_i, b):
        pltpu.async_copy(
            w_hbm.at[pl.ds(ebase + chunk_i * CHUNK, CHUNK)], wc[b], sem_w[b])

    def wait_w(b):
        pltpu.make_async_copy(
            w_hbm.at[pl.ds(ebase, CHUNK)], wc[b], sem_w[b]).wait()

    def start_scatter(bi, bm):
        pltpu.async_copy(msg[bm], agg_sh.at[dstb[bi]], sem_s[bm], add=True)

    def wait_scatter(bi, bm):
        pltpu.make_async_copy(msg[bm], agg_sh.at[dstb[bi]], sem_s[bm]).wait()

    def multiply(b):
        def _mrow(r, cc):
            for j in range(D // 16):
                sl = pl.ds(j * 16, 16)
                msg[b][r, sl] = rows[b][r, sl] * wc[b][r, sl]
            return cc

        lax.fori_loop(0, CHUNK, _mrow, 0)

    def _zrow(i, carry):
        for j in range(D // 16):
            z_v[i, pl.ds(j * 16, 16)] = jnp.zeros((16,), jnp.float32)
        return carry

    lax.fori_loop(0, ZROWS, _zrow, 0)
    row0 = s * DUMP_ROWS
    for j in range(DUMP_ROWS // ZROWS):
        pltpu.sync_copy(z_v, agg_sh.at[pl.ds(row0 + j * ZROWS, ZROWS)])

    @pl.when(s == NS - 1)
    def _zero_tail():
        pltpu.sync_copy(
            z_v.at[pl.ds(0, TAIL_ROWS)],
            agg_sh.at[pl.ds(NS * DUMP_ROWS, TAIL_ROWS)],
        )

    plsc.subcore_barrier()

    def step_a(j, g=None):
        bi, br = j % NIDX, j % NROW
        wait_gather(bi, br)
        wait_w(br)
        if g is None or j >= NROW:
            wait_scatter((j - NROW) % NIDX, br)
        else:
            @pl.when(g > 0)
            def _ws():
                wait_scatter((j - NROW) % NIDX, br)
        wait_dst(bi)
        multiply(br)
        start_scatter(bi, br)

    def step_b(j, i2):
        bi2, br = (j + 2) % NIDX, j % NROW
        wait_src(bi2)
        start_dst(i2, bi2)
        start_gather(bi2, br)
        start_w(i2, br)

    for k in range(NIDX):
        start_src(k, k)
    for k in range(NROW):
        wait_src(k)
        start_dst(k, k)
        start_gather(k, k)
        start_w(k, k)

    if n_chunks % GROUP == 0:
        n_main, ep = n_chunks // GROUP - 1, GROUP
    else:
        n_main, ep = n_chunks // GROUP, n_chunks % GROUP
    assert ep >= NIDX

    def _group(g, carry):
        for j in range(GROUP):
            i = g * GROUP + j
            step_a(j, g)
            step_b(j, i + 2)
            start_src(i + NIDX, j % NIDX)
        return carry

    lax.fori_loop(0, n_main, _group, 0)

    i0 = n_main * GROUP
    for j in range(ep):
        step_a(j)
        if i0 + j + 2 < n_chunks:
            step_b(j, i0 + j + 2)
        if i0 + j + NIDX < n_chunks:
            start_src(i0 + j + NIDX, j % NIDX)
    for bm in range(NROW):
        wait_scatter((ep - NROW + bm) % NIDX, bm)

    plsc.subcore_barrier()

    pltpu.sync_copy(
        agg_sh.at[pl.ds(row0, DUMP_ROWS)],
        out_hbm.at[c, pl.ds(row0, DUMP_ROWS)],
    )

    @pl.when(s == NS - 1)
    def _dump_tail():
        pltpu.sync_copy(
            agg_sh.at[pl.ds(NS * DUMP_ROWS, TAIL_ROWS)],
            out_hbm.at[c, pl.ds(NS * DUMP_ROWS, TAIL_ROWS)],
        )


@functools.lru_cache(maxsize=4)
def _edge_aggregate_kernel(n_edges):
    edges_per_tile = n_edges // NW
    n_chunks = edges_per_tile // CHUNK
    scratch = (
        [pltpu.VMEM((CHUNK,), jnp.int32) for _ in range(NIDX)]
        + [pltpu.VMEM((CHUNK,), jnp.int32) for _ in range(NIDX)]
        + [pltpu.VMEM((CHUNK, D), jnp.float32) for _ in range(NROW)]
        + [pltpu.VMEM((CHUNK, D), jnp.float32) for _ in range(NROW)]
        + [pltpu.VMEM((CHUNK, D), jnp.float32) for _ in range(NROW)]
        + [pltpu.VMEM((ZROWS, D), jnp.float32)]
        + [pltpu.VMEM_SHARED((N_NODES, D), jnp.float32)]
        + [pltpu.SemaphoreType.DMA for _ in range(2 * NIDX + 3 * NROW)]
    )
    return pl.kernel(
        functools.partial(_edge_aggregate_body, edges_per_tile, n_chunks),
        out_type=jax.ShapeDtypeStruct((NC, N_NODES, D), jnp.float32),
        mesh=_SC_MESH,
        scratch_types=scratch,
    )


def _combine_body(a0_ref, a1_ref, a2_ref, a3_ref, h_ref, ws_ref, wa_ref,
                  wo_ref, bo_ref, o_ref):
    agg = (a0_ref[...] + a1_ref[...]) + (a2_ref[...] + a3_ref[...])
    t = jnp.dot(agg, ws_ref[...], preferred_element_type=jnp.float32) + jnp.dot(
        h_ref[...], wa_ref[...], preferred_element_type=jnp.float32
    )
    t = _silu(t)
    o_ref[...] = _silu(
        jnp.dot(t, wo_ref[...], preferred_element_type=jnp.float32) + bo_ref[...]
    )


def _combine(a0, a1, a2, a3, h, w_self, w_attr, w_out, b_out):
    grid = (N_NODES // NODE_BLK,)
    blk = lambda i: (i, 0)
    full = lambda i: (0, 0)
    return pl.pallas_call(
        _combine_body,
        grid=grid,
        in_specs=[
            pl.BlockSpec((NODE_BLK, D), blk),
            pl.BlockSpec((NODE_BLK, D), blk),
            pl.BlockSpec((NODE_BLK, D), blk),
            pl.BlockSpec((NODE_BLK, D), blk),
            pl.BlockSpec((NODE_BLK, D), blk),
            pl.BlockSpec((D, D), full),
            pl.BlockSpec((D, D), full),
            pl.BlockSpec((D, D), full),
            pl.BlockSpec((1, D), full),
        ],
        out_specs=pl.BlockSpec((NODE_BLK, D), blk),
        out_shape=jax.ShapeDtypeStruct((N_NODES, D), jnp.float32),
    )(a0, a1, a2, a3, h, w_self, w_attr, w_out, b_out)


def kernel(x, edge_index, edge_vec, W_emb1, b_emb1, W_emb2, b_emb2,
           Wr1, br1, Wr2, br2, Wr3, W_self, W_attr, W_out, b_out):
    h = _node_embed(x, W_emb1, b_emb1.reshape(1, D), W_emb2, b_emb2.reshape(1, D))
    ei = edge_index.astype(jnp.int32)
    evt = edge_vec.T
    e2 = N_EDGES // 2
    br1r = br1.reshape(1, 64)
    br2r = br2.reshape(1, 64)
    w_a = _radial(evt[:, :e2], e2, Wr1, br1r, Wr2, br2r, Wr3)
    parts_a = _edge_aggregate_kernel(e2)(h, w_a, ei[0, :e2], ei[1, :e2])
    w_b = _radial(evt[:, e2:], e2, Wr1, br1r, Wr2, br2r, Wr3)
    parts_b = _edge_aggregate_kernel(e2)(h, w_b, ei[0, e2:], ei[1, e2:])
    return _combine(parts_a[0], parts_a[1], parts_b[0], parts_b[1], h,
                    W_self, W_attr, W_out, b_out.reshape(1, D))


# speedup vs baseline: 1.1041x; 1.0272x over previous
import functools

import jax
import jax.numpy as jnp
from jax import lax
from jax.experimental import pallas as pl
from jax.experimental.pallas import tpu as pltpu
from jax.experimental.pallas import tpu_sc as plsc

N_NODES = 10000
N_EDGES = 320000
D = 128
N_BASIS = 16
R_MAX = 5.0

NC = 2
NS = 16
NW = NC * NS
EDGES_PER_TILE = N_EDGES // NW
CHUNK = 40
N_CHUNKS = EDGES_PER_TILE // CHUNK
DUMP_ROWS = 624
TAIL_ROWS = N_NODES - NS * DUMP_ROWS
ZROWS = 52

NODE_BLK = 1000
EDGE_BLK = 3200


def _silu(v):
    return v * (1.0 / (1.0 + jnp.exp(-v)))


def _pack_bf16_pairs(a, b):
    ua = lax.bitcast_convert_type(
        a.astype(jnp.bfloat16).astype(jnp.float32), jnp.int32)
    ub = lax.bitcast_convert_type(
        b.astype(jnp.bfloat16).astype(jnp.float32), jnp.int32)
    return (ub & (-65536)) | lax.shift_right_logical(ua, 16)


def _node_embed_body(x_ref, w1_ref, b1_ref, w2_ref, b2_ref, h_ref):
    t = jnp.dot(x_ref[...], w1_ref[...], preferred_element_type=jnp.float32)
    t = _silu(t + b1_ref[...])
    h_ref[...] = (
        jnp.dot(t, w2_ref[...], preferred_element_type=jnp.float32) + b2_ref[...]
    )


def _node_embed(x, w1, b1, w2, b2):
    grid = (N_NODES // NODE_BLK,)
    return pl.pallas_call(
        _node_embed_body,
        grid=grid,
        in_specs=[
            pl.BlockSpec((NODE_BLK, D), lambda i: (i, 0)),
            pl.BlockSpec((D, D), lambda i: (0, 0)),
            pl.BlockSpec((1, D), lambda i: (0, 0)),
            pl.BlockSpec((D, D), lambda i: (0, 0)),
            pl.BlockSpec((1, D), lambda i: (0, 0)),
        ],
        out_specs=pl.BlockSpec((NODE_BLK, D), lambda i: (i, 0)),
        out_shape=jax.ShapeDtypeStruct((N_NODES, D), jnp.float32),
    )(x, w1, b1, w2, b2)


def _radial_body(evt_ref, wr1_ref, br1_ref, wr2_ref, br2_ref, wr3_ref, w_ref):
    evt = evt_ref[...]
    r2 = jnp.sum(evt * evt, axis=0, keepdims=True) + 1e-12
    r = jnp.sqrt(r2)
    ru = jnp.clip(r * (1.0 / R_MAX), 0.0, 1.0)
    t_ = jnp.pi * (ru - 0.5)
    t2 = t_ * t_
    sin_t = t_ * (1.0 + t2 * (-1.0 / 6.0 + t2 * (1.0 / 120.0 - t2 / 5040.0)))
    cutoff = 0.5 * (1.0 - sin_t)
    step = R_MAX / (N_BASIS - 1)
    centers = lax.broadcasted_iota(jnp.int32, (N_BASIS, 1), 0).astype(
        jnp.float32) * step
    width = R_MAX / N_BASIS
    diff = jnp.broadcast_to(r, (N_BASIS, r.shape[1])) - centers
    basis_t = jnp.exp(-(diff * diff) * (1.0 / (width * width)))
    basis_t = basis_t * cutoff
    bf = jnp.bfloat16
    t = _silu(
        lax.dot_general(
            basis_t.astype(bf), wr1_ref[...].astype(bf),
            dimension_numbers=(((0,), (0,)), ((), ())),
            preferred_element_type=jnp.float32,
        )
        + br1_ref[...]
    )
    t = _silu(
        jnp.dot(t.astype(bf), wr2_ref[...].astype(bf),
                preferred_element_type=jnp.float32) + br2_ref[...]
    )
    w_ref[...] = jnp.dot(t.astype(bf), wr3_ref[...].astype(bf),
                         preferred_element_type=jnp.float32)


def _radial(edge_vec_t, n_edges, wr1, br1, wr2, br2, wr3):
    grid = (n_edges // EDGE_BLK,)
    return pl.pallas_call(
        _radial_body,
        grid=grid,
        in_specs=[
            pl.BlockSpec((3, EDGE_BLK), lambda i: (0, i)),
            pl.BlockSpec((N_BASIS, 64), lambda i: (0, 0)),
            pl.BlockSpec((1, 64), lambda i: (0, 0)),
            pl.BlockSpec((64, 64), lambda i: (0, 0)),
            pl.BlockSpec((1, 64), lambda i: (0, 0)),
            pl.BlockSpec((64, D), lambda i: (0, 0)),
        ],
        out_specs=pl.BlockSpec((EDGE_BLK, D), lambda i: (i, 0)),
        out_shape=jax.ShapeDtypeStruct((n_edges, D), jnp.float32),
    )(edge_vec_t, wr1, br1, wr2, br2, wr3)


_SC_MESH = plsc.VectorSubcoreMesh(
    core_axis_name="c", subcore_axis_name="s", num_cores=NC, num_subcores=NS
)


NIDX = 5
NROW = 2
GROUP = 10


def _edge_aggregate_body(edges_per_tile, n_chunks,
                         h_hbm, w_hbm, src_hbm, dst_hbm, out_hbm, *scr):
    srcb = scr[0:NIDX]
    dstb = scr[NIDX:2 * NIDX]
    rows = scr[2 * NIDX:2 * NIDX + NROW]
    wc = scr[2 * NIDX + NROW:2 * NIDX + 2 * NROW]
    msg = scr[2 * NIDX + 2 * NROW:2 * NIDX + 3 * NROW]
    z_v = scr[2 * NIDX + 3 * NROW]
    agg_sh = scr[2 * NIDX + 3 * NROW + 1]
    sems = scr[2 * NIDX + 3 * NROW + 2:]
    sem_si = sems[0:NIDX]
    sem_di = sems[NIDX:2 * NIDX]
    sem_g = sems[2 * NIDX:2 * NIDX + NROW]
    sem_w = sems[2 * NIDX + NROW:2 * NIDX + 2 * NROW]
    sem_s = sems[2 * NIDX + 2 * NROW:2 * NIDX + 3 * NROW]

    c = lax.axis_index("c")
    s = lax.axis_index("s")
    wid = c * NS + s
    ebase = wid * edges_per_tile

    def start_src(chunk_i, b):
        pltpu.async_copy(
            src_hbm.at[pl.ds(ebase + chunk_i * CHUNK, CHUNK)], srcb[b],
            sem_si[b])

    def wait_src(b):
        pltpu.make_async_copy(
            src_hbm.at[pl.ds(ebase, CHUNK)], srcb[b], sem_si[b]).wait()

    def start_dst(chunk_i, b):
        pltpu.async_copy(
            dst_hbm.at[pl.ds(ebase + chunk_i * CHUNK, CHUNK)], dstb[b],
            sem_di[b])

    def wait_dst(b):
        pltpu.make_async_copy(
            dst_hbm.at[pl.ds(ebase, CHUNK)], dstb[b], sem_di[b]).wait()

    def start_gather(bi, br):
        pltpu.async_copy(h_hbm.at[srcb[bi]], rows[br], sem_g[br])

    def wait_gather(bi, br):
        pltpu.make_async_copy(h_hbm.at[srcb[bi]], rows[br], sem_g[br]).wait()

    def start_w(chunk_i, b):
        pltpu.async_copy(
            w_hbm.at[pl.ds(ebase + chunk_i * CHUNK, CHUNK)], wc[b], sem_w[b])

    def wait_w(b):
        pltpu.make_async_copy(
            w_hbm.at[pl.ds(ebase, CHUNK)], wc[b], sem_w[b]).wait()

    def start_scatter(bi, bm):
        pltpu.async_copy(msg[bm], agg_sh.at[dstb[bi]], sem_s[bm], add=True)

    def wait_scatter(bi, bm):
        pltpu.make_async_copy(msg[bm], agg_sh.at[dstb[bi]], sem_s[bm]).wait()

    def multiply(b):
        def _mrow(r, cc):
            for j in range(D // 16):
                sl = pl.ds(j * 16, 16)
                msg[b][r, sl] = rows[b][r, sl] * wc[b][r, sl]
            return cc

        lax.fori_loop(0, CHUNK, _mrow, 0)

    def _zrow(i, carry):
        for j in range(D // 16):
            z_v[i, pl.ds(j * 16, 16)] = jnp.zeros((16,), jnp.float32)
        return carry

    lax.fori_loop(0, ZROWS, _zrow, 0)
    row0 = s * DUMP_ROWS
    for j in range(DUMP_ROWS // ZROWS):
        pltpu.sync_copy(z_v, agg_sh.at[pl.ds(row0 + j * ZROWS, ZROWS)])

    @pl.when(s == NS - 1)
    def _zero_tail():
        pltpu.sync_copy(
            z_v.at[pl.ds(0, TAIL_ROWS)],
            agg_sh.at[pl.ds(NS * DUMP_ROWS, TAIL_ROWS)],
        )

    plsc.subcore_barrier()

    def step_a(j, g=None):
        bi, br = j % NIDX, j % NROW
        wait_gather(bi, br)
        wait_w(br)
        if g is None or j >= NROW:
            wait_scatter((j - NROW) % NIDX, br)
        else:
            @pl.when(g > 0)
            def _ws():
                wait_scatter((j - NROW) % NIDX, br)
        wait_dst(bi)
        multiply(br)
        start_scatter(bi, br)

    def step_b(j, i2):
        bi2, br = (j + 2) % NIDX, j % NROW
        wait_src(bi2)
        start_dst(i2, bi2)
        start_gather(bi2, br)
        start_w(i2, br)

    for k in range(NIDX):
        start_src(k, k)
    for k in range(NROW):
        wait_src(k)
        start_dst(k, k)
        start_gather(k, k)
        start_w(k, k)

    if n_chunks % GROUP == 0:
        n_main, ep = n_chunks // GROUP - 1, GROUP
    else:
        n_main, ep = n_chunks // GROUP, n_chunks % GROUP
    assert ep >= NIDX

    def _group(g, carry):
        for j in range(GROUP):
            i = g * GROUP + j
            step_a(j, g)
            step_b(j, i + 2)
            start_src(i + NIDX, j % NIDX)
        return carry

    lax.fori_loop(0, n_main, _group, 0)

    i0 = n_main * GROUP
    for j in range(ep):
        step_a(j)
        if i0 + j + 2 < n_chunks:
            step_b(j, i0 + j + 2)
        if i0 + j + NIDX < n_chunks:
            start_src(i0 + j + NIDX, j % NIDX)
    for bm in range(NROW):
        wait_scatter((ep - NROW + bm) % NIDX, bm)

    plsc.subcore_barrier()

    pltpu.sync_copy(
        agg_sh.at[pl.ds(row0, DUMP_ROWS)],
        out_hbm.at[c, pl.ds(row0, DUMP_ROWS)],
    )

    @pl.when(s == NS - 1)
    def _dump_tail():
        pltpu.sync_copy(
            agg_sh.at[pl.ds(NS * DUMP_ROWS, TAIL_ROWS)],
            out_hbm.at[c, pl.ds(NS * DUMP_ROWS, TAIL_ROWS)],
        )


@functools.lru_cache(maxsize=4)
def _edge_aggregate_kernel(n_edges):
    edges_per_tile = n_edges // NW
    n_chunks = edges_per_tile // CHUNK
    scratch = (
        [pltpu.VMEM((CHUNK,), jnp.int32) for _ in range(NIDX)]
        + [pltpu.VMEM((CHUNK,), jnp.int32) for _ in range(NIDX)]
        + [pltpu.VMEM((CHUNK, D), jnp.float32) for _ in range(NROW)]
        + [pltpu.VMEM((CHUNK, D), jnp.float32) for _ in range(NROW)]
        + [pltpu.VMEM((CHUNK, D), jnp.float32) for _ in range(NROW)]
        + [pltpu.VMEM((ZROWS, D), jnp.float32)]
        + [pltpu.VMEM_SHARED((N_NODES, D), jnp.float32)]
        + [pltpu.SemaphoreType.DMA for _ in range(2 * NIDX + 3 * NROW)]
    )
    return pl.kernel(
        functools.partial(_edge_aggregate_body, edges_per_tile, n_chunks),
        out_type=jax.ShapeDtypeStruct((NC, N_NODES, D), jnp.float32),
        mesh=_SC_MESH,
        scratch_types=scratch,
    )


def _combine_body(a0_ref, a1_ref, a2_ref, a3_ref, h_ref, ws_ref, wa_ref,
                  wo_ref, bo_ref, o_ref):
    agg = (a0_ref[...] + a1_ref[...]) + (a2_ref[...] + a3_ref[...])
    t = jnp.dot(agg, ws_ref[...], preferred_element_type=jnp.float32) + jnp.dot(
        h_ref[...], wa_ref[...], preferred_element_type=jnp.float32
    )
    t = _silu(t)
    o_ref[...] = _silu(
        jnp.dot(t, wo_ref[...], preferred_element_type=jnp.float32) + bo_ref[...]
    )


def _combine(a0, a1, a2, a3, h, w_self, w_attr, w_out, b_out):
    grid = (N_NODES // NODE_BLK,)
    blk = lambda i: (i, 0)
    full = lambda i: (0, 0)
    return pl.pallas_call(
        _combine_body,
        grid=grid,
        in_specs=[
            pl.BlockSpec((NODE_BLK, D), blk),
            pl.BlockSpec((NODE_BLK, D), blk),
            pl.BlockSpec((NODE_BLK, D), blk),
            pl.BlockSpec((NODE_BLK, D), blk),
            pl.BlockSpec((NODE_BLK, D), blk),
            pl.BlockSpec((D, D), full),
            pl.BlockSpec((D, D), full),
            pl.BlockSpec((D, D), full),
            pl.BlockSpec((1, D), full),
        ],
        out_specs=pl.BlockSpec((NODE_BLK, D), blk),
        out_shape=jax.ShapeDtypeStruct((N_NODES, D), jnp.float32),
    )(a0, a1, a2, a3, h, w_self, w_attr, w_out, b_out)


def kernel(x, edge_index, edge_vec, W_emb1, b_emb1, W_emb2, b_emb2,
           Wr1, br1, Wr2, br2, Wr3, W_self, W_attr, W_out, b_out):
    h = _node_embed(x, W_emb1, b_emb1.reshape(1, D), W_emb2, b_emb2.reshape(1, D))
    ei = edge_index.astype(jnp.int32)
    evt = edge_vec.T
    e2 = N_EDGES // 2
    br1r = br1.reshape(1, 64)
    br2r = br2.reshape(1, 64)
    w_a = _radial(evt[:, :e2], e2, Wr1, br1r, Wr2, br2r, Wr3)
    parts_a = _edge_aggregate_kernel(e2)(h, w_a, ei[0, :e2], ei[1, :e2])
    w_b = _radial(evt[:, e2:], e2, Wr1, br1r, Wr2, br2r, Wr3)
    parts_b = _edge_aggregate_kernel(e2)(h, w_b, ei[0, e2:], ei[1, e2:])
    return _combine(parts_a[0], parts_a[1], parts_b[0], parts_b[1], h,
                    W_self, W_attr, W_out, b_out.reshape(1, D))
